# baseline (speedup 1.0000x reference)
import functools

import jax
import jax.numpy as jnp
from jax import lax
from jax.experimental import pallas as pl
from jax.experimental.pallas import tpu as pltpu
from jax.experimental.pallas import tpu_sc as plsc

N_NODES = 50000
FEAT = 896
HIDDEN = 64
NUM_CLASSES = 2
BATCH = 16384

ZCOL = 16
ZSTRIDE = 128
ROWS_BLK = 4096

_NC, _NS = 2, 16
_NW = _NC * _NS
_CH = 128
_NCH = BATCH // _NW // _CH
_L = 16


def _mlp_body(a_ref, w1_ref, w2_ref, z_ref):
    h = jnp.dot(a_ref[...].astype(jnp.bfloat16), w1_ref[...],
                preferred_element_type=jnp.float32)
    h = jnp.where(h >= 0, h, 0.3 * h)
    z_ref[...] = jnp.dot(h, w2_ref[...], preferred_element_type=jnp.float32)


def _mlp_all_rows(agg_table, w1, w2pad):
    grid = (N_NODES + ROWS_BLK - 1) // ROWS_BLK
    return pl.pallas_call(
        _mlp_body,
        grid=(grid,),
        in_specs=[
            pl.BlockSpec((ROWS_BLK, FEAT), lambda i: (i, 0)),
            pl.BlockSpec((FEAT, HIDDEN), lambda i: (0, 0)),
            pl.BlockSpec((HIDDEN, ZSTRIDE), lambda i: (0, 0)),
        ],
        out_specs=pl.BlockSpec((ROWS_BLK, ZSTRIDE), lambda i: (i, 0)),
        out_shape=jax.ShapeDtypeStruct((N_NODES, ZSTRIDE), jnp.float32),
        compiler_params=pltpu.CompilerParams(
            dimension_semantics=("parallel",),
            vmem_limit_bytes=120 * 1024 * 1024,
        ),
    )(agg_table, w1, w2pad)


_ECH = BATCH * NUM_CLASSES // _NW // _CH


@functools.cache
def _sc_gather_kernel():
    n_per_w = BATCH // _NW
    e_per_w = n_per_w * NUM_CLASSES

    @functools.partial(
        pl.kernel,
        out_type=jax.ShapeDtypeStruct((_NW, e_per_w), jnp.float32),
        mesh=plsc.VectorSubcoreMesh(
            core_axis_name="c", subcore_axis_name="s", num_cores=_NC),
        scratch_types=[
            pltpu.VMEM((n_per_w,), jnp.int32),
            pltpu.VMEM((e_per_w,), jnp.int32),
            pltpu.VMEM((e_per_w,), jnp.float32),
            pltpu.SemaphoreType.DMA,
        ],
    )
    def _sc_gather(z_hbm, idx_hbm, out_hbm, idx_v, eidx_v, out_v, sem):
        wid = lax.axis_index("s") * _NC + lax.axis_index("c")
        pltpu.sync_copy(idx_hbm.at[wid], idx_v)
        for g in range(n_per_w // _L):
            node = idx_v[pl.ds(g * _L, _L)]
            base = jnp.left_shift(node, 7)
            for c in range(NUM_CLASSES):
                eidx_v[pl.ds(c * n_per_w + g * _L, _L)] = base + c
        copies = [
            pltpu.async_copy(
                z_hbm.at[eidx_v.at[pl.ds(j * _CH, _CH)]],
                out_v.at[pl.ds(j * _CH, _CH)], sem)
            for j in range(e_per_w // _CH)
        ]
        for c in copies:
            c.wait()
        pltpu.sync_copy(out_v, out_hbm.at[wid])

    return _sc_gather


def kernel(nodes, agg_table, weight_model, weight_model2):
    w2pad = jnp.zeros((HIDDEN, ZSTRIDE), jnp.float32).at[:, :NUM_CLASSES].set(
        weight_model2)
    z8 = _mlp_all_rows(agg_table, weight_model.astype(jnp.bfloat16), w2pad)
    idx = nodes.reshape(_NW, BATCH // _NW)
    g = _sc_gather_kernel()(z8.reshape(N_NODES * ZSTRIDE), idx)
    n_per_w = BATCH // _NW
    return (g.reshape(_NW, NUM_CLASSES, n_per_w)
            .transpose(0, 2, 1).reshape(BATCH, NUM_CLASSES))


# speedup vs baseline: 1.0002x; 1.0002x over previous
import functools

import jax
import jax.numpy as jnp
from jax import lax
from jax.experimental import pallas as pl
from jax.experimental.pallas import tpu as pltpu
from jax.experimental.pallas import tpu_sc as plsc

N_NODES = 50000
FEAT = 896
HIDDEN = 64
NUM_CLASSES = 2
BATCH = 16384

ZCOL = 16
ZSTRIDE = 128
ROWS_BLK = 4096

_NC, _NS = 2, 16
_NW = _NC * _NS
_CH = 128
_NCH = BATCH // _NW // _CH
_L = 16


def _mlp_body(a_ref, w1_ref, w2_ref, z_ref):
    h = jnp.dot(a_ref[...].astype(jnp.bfloat16), w1_ref[...],
                preferred_element_type=jnp.float32)
    h = jnp.where(h >= 0, h, 0.3 * h)
    z_ref[...] = jnp.dot(h.astype(jnp.bfloat16), w2_ref[...],
                         preferred_element_type=jnp.float32)


def _mlp_all_rows(agg_table, w1, w2pad):
    grid = (N_NODES + ROWS_BLK - 1) // ROWS_BLK
    return pl.pallas_call(
        _mlp_body,
        grid=(grid,),
        in_specs=[
            pl.BlockSpec((ROWS_BLK, FEAT), lambda i: (i, 0)),
            pl.BlockSpec((FEAT, HIDDEN), lambda i: (0, 0)),
            pl.BlockSpec((HIDDEN, ZSTRIDE), lambda i: (0, 0)),
        ],
        out_specs=pl.BlockSpec((ROWS_BLK, ZSTRIDE), lambda i: (i, 0)),
        out_shape=jax.ShapeDtypeStruct((N_NODES, ZSTRIDE), jnp.float32),
        compiler_params=pltpu.CompilerParams(
            dimension_semantics=("parallel",),
            vmem_limit_bytes=120 * 1024 * 1024,
        ),
    )(agg_table, w1, w2pad)


_ECH = BATCH * NUM_CLASSES // _NW // _CH


@functools.cache
def _sc_gather_kernel():
    n_per_w = BATCH // _NW
    e_per_w = n_per_w * NUM_CLASSES

    @functools.partial(
        pl.kernel,
        out_type=jax.ShapeDtypeStruct((_NW, e_per_w), jnp.float32),
        mesh=plsc.VectorSubcoreMesh(
            core_axis_name="c", subcore_axis_name="s", num_cores=_NC),
        scratch_types=[
            pltpu.VMEM((n_per_w,), jnp.int32),
            pltpu.VMEM((e_per_w,), jnp.int32),
            pltpu.VMEM((e_per_w,), jnp.float32),
            pltpu.SemaphoreType.DMA,
        ],
    )
    def _sc_gather(z_hbm, idx_hbm, out_hbm, idx_v, eidx_v, out_v, sem):
        wid = lax.axis_index("s") * _NC + lax.axis_index("c")
        pltpu.sync_copy(idx_hbm.at[wid], idx_v)
        for g in range(n_per_w // _L):
            node = idx_v[pl.ds(g * _L, _L)]
            base = jnp.left_shift(node, 7)
            for c in range(NUM_CLASSES):
                eidx_v[pl.ds(c * n_per_w + g * _L, _L)] = base + c
        copies = [
            pltpu.async_copy(
                z_hbm.at[eidx_v.at[pl.ds(j * _CH, _CH)]],
                out_v.at[pl.ds(j * _CH, _CH)], sem)
            for j in range(e_per_w // _CH)
        ]
        for c in copies:
            c.wait()
        pltpu.sync_copy(out_v, out_hbm.at[wid])

    return _sc_gather


def kernel(nodes, agg_table, weight_model, weight_model2):
    w2pad = jnp.zeros((HIDDEN, ZSTRIDE), jnp.float32).at[:, :NUM_CLASSES].set(
        weight_model2).astype(jnp.bfloat16)
    z8 = _mlp_all_rows(agg_table, weight_model.astype(jnp.bfloat16), w2pad)
    idx = nodes.reshape(_NW, BATCH // _NW)
    g = _sc_gather_kernel()(z8.reshape(N_NODES * ZSTRIDE), idx)
    n_per_w = BATCH // _NW
    return (g.reshape(_NW, NUM_CLASSES, n_per_w)
            .transpose(0, 2, 1).reshape(BATCH, NUM_CLASSES))
